# Initial kernel scaffold; baseline (speedup 1.0000x reference)
import jax
import jax.numpy as jnp
from jax.experimental import pallas as pl


def kernel(outputs1, outputs2, available):
    raise NotImplementedError("write your pallas kernel here")


# speedup vs baseline: 1.7506x; 1.7506x over previous
import numpy as np
import jax
import jax.numpy as jnp
from jax import lax
from jax.experimental import pallas as pl
from jax.experimental.pallas import tpu as pltpu

_B, _D = 8192, 2048
_RB = 128

_R0 = (13, 15, 26, 6)
_R1 = (17, 29, 16, 24)


def _threefry2x32_np(k0, k1, x0, x1):
    x0 = np.uint32(x0); x1 = np.uint32(x1)
    ks0, ks1 = np.uint32(k0), np.uint32(k1)
    ks2 = np.uint32(ks0 ^ ks1 ^ np.uint32(0x1BD11BDA))
    def rotl(x, r):
        return np.uint32((np.uint32(x << np.uint32(r))) | np.uint32(x >> np.uint32(32 - r)))
    x0 = np.uint32(x0 + ks0); x1 = np.uint32(x1 + ks1)
    sched = [(_R0, ks1, np.uint32(ks2 + 1)), (_R1, ks2, np.uint32(ks0 + 2)),
             (_R0, ks0, np.uint32(ks1 + 3)), (_R1, ks1, np.uint32(ks2 + 4)),
             (_R0, ks2, np.uint32(ks0 + 5))]
    for rots, i0, i1 in sched:
        for r in rots:
            x0 = np.uint32(x0 + x1); x1 = rotl(x1, r); x1 = np.uint32(x1 ^ x0)
        x0 = np.uint32(x0 + i0); x1 = np.uint32(x1 + i1)
    return x0, x1


def _fold_in_42(i):
    with np.errstate(over="ignore"):
        o0, o1 = _threefry2x32_np(0, 42, 0, i)
    return int(np.int32(o0)), int(np.int32(o1))


_KD1 = _fold_in_42(1)
_KD2 = _fold_in_42(2)


def _i32(x):
    return jnp.int32(np.int32(np.uint32(x)))


def _tf_bits23(kd, cnt):
    k0 = np.uint32(np.int32(kd[0]))
    k1 = np.uint32(np.int32(kd[1]))
    ks2 = np.uint32(k0 ^ k1 ^ np.uint32(0x1BD11BDA))

    def rotl(x, r):
        return lax.shift_left(x, jnp.int32(r)) | lax.shift_right_logical(
            x, jnp.int32(32 - r))

    x0 = jnp.full_like(cnt, _i32(k0))
    x1 = cnt + _i32(k1)
    sched = ((_R0, k1, np.uint32(ks2 + 1)), (_R1, ks2, np.uint32(k0 + 2)),
             (_R0, k0, np.uint32(k1 + 3)), (_R1, k1, np.uint32(ks2 + 4)),
             (_R0, ks2, np.uint32(k0 + 5)))
    for rots, i0, i1 in sched:
        for r in rots:
            x0 = x0 + x1
            x1 = rotl(x1, r)
            x1 = x1 ^ x0
        x0 = x0 + _i32(i0)
        x1 = x1 + _i32(i1)
    return lax.shift_right_logical(x0 ^ x1, jnp.int32(9))


def _body(avail_ref, x1_ref, x2_ref, out_ref, out1_ref):
    i = pl.program_id(0)
    r0 = i * _RB
    row = lax.broadcasted_iota(jnp.int32, (_RB, _D), 0)
    col = lax.broadcasted_iota(jnp.int32, (_RB, _D), 1)
    flat = (r0 + row) * _D + col

    base = flat * 4
    best = jnp.full((_RB, _D), -1, jnp.int32)
    val = jnp.zeros((_RB, _D), jnp.float32)
    for m in range(4):
        bits = _tf_bits23(_KD1, base + m)
        take = (bits > best) & (avail_ref[m] != 0)
        best = jnp.where(take, bits, best)
        val = jnp.where(take, x1_ref[m], val)
    out1_ref[...] = val

    base = flat * 5
    best = jnp.full((_RB, _D), -1, jnp.int32)
    val2 = jnp.zeros((_RB, _D), jnp.float32)
    for m in range(5):
        bits = _tf_bits23(_KD2, base + m)
        take = (bits > best) & (avail_ref[m] != 0)
        best = jnp.where(take, bits, best)
        val2 = jnp.where(take, x2_ref[m] if m < 4 else val, val2)
    out_ref[...] = val2


def kernel(outputs1, outputs2, available):
    avail_mask = (available != 0.0).astype(jnp.int32)
    grid = (_B // _RB,)
    out, out1 = pl.pallas_call(
        _body,
        grid=grid,
        in_specs=[
            pl.BlockSpec(memory_space=pltpu.SMEM),
            pl.BlockSpec((4, _RB, _D), lambda i: (0, i, 0)),
            pl.BlockSpec((4, _RB, _D), lambda i: (0, i, 0)),
        ],
        out_specs=[
            pl.BlockSpec((_RB, _D), lambda i: (i, 0)),
            pl.BlockSpec((_RB, _D), lambda i: (i, 0)),
        ],
        out_shape=[
            jax.ShapeDtypeStruct((_B, _D), jnp.float32),
            jax.ShapeDtypeStruct((_B, _D), jnp.float32),
        ],
        compiler_params=pltpu.CompilerParams(
            dimension_semantics=("arbitrary",),
        ),
    )(avail_mask, outputs1, outputs2)
    return (out, out1)


# speedup vs baseline: 1.9235x; 1.0988x over previous
import functools

import numpy as np
import jax
import jax.numpy as jnp
from jax import lax
from jax.experimental import pallas as pl
from jax.experimental.pallas import tpu as pltpu
from jax.experimental.pallas import tpu_sc as plsc

_B, _D = 8192, 2048
_RB = 128
_SC_ROWS = 1024
_NW = 32
_RC = 2

_R0 = (13, 15, 26, 6)
_R1 = (17, 29, 16, 24)


def _threefry2x32_np(k0, k1, x0, x1):
    x0 = np.uint32(x0); x1 = np.uint32(x1)
    ks0, ks1 = np.uint32(k0), np.uint32(k1)
    ks2 = np.uint32(ks0 ^ ks1 ^ np.uint32(0x1BD11BDA))
    def rotl(x, r):
        return np.uint32((np.uint32(x << np.uint32(r))) | np.uint32(x >> np.uint32(32 - r)))
    x0 = np.uint32(x0 + ks0); x1 = np.uint32(x1 + ks1)
    sched = [(_R0, ks1, np.uint32(ks2 + 1)), (_R1, ks2, np.uint32(ks0 + 2)),
             (_R0, ks0, np.uint32(ks1 + 3)), (_R1, ks1, np.uint32(ks2 + 4)),
             (_R0, ks2, np.uint32(ks0 + 5))]
    for rots, i0, i1 in sched:
        for r in rots:
            x0 = np.uint32(x0 + x1); x1 = rotl(x1, r); x1 = np.uint32(x1 ^ x0)
        x0 = np.uint32(x0 + i0); x1 = np.uint32(x1 + i1)
    return x0, x1


def _fold_in_42(i):
    with np.errstate(over="ignore"):
        o0, o1 = _threefry2x32_np(0, 42, 0, i)
    return int(np.int32(o0)), int(np.int32(o1))


_KD1 = _fold_in_42(1)
_KD2 = _fold_in_42(2)


def _i32(x):
    return jnp.int32(np.int32(np.uint32(x)))


def _tf_bits23(kd, cnt):
    k0 = np.uint32(np.int32(kd[0]))
    k1 = np.uint32(np.int32(kd[1]))
    ks2 = np.uint32(k0 ^ k1 ^ np.uint32(0x1BD11BDA))

    def rotl(x, r):
        return lax.shift_left(x, jnp.int32(r)) | lax.shift_right_logical(
            x, jnp.int32(32 - r))

    x0 = jnp.full_like(cnt, _i32(k0))
    x1 = cnt + _i32(k1)
    sched = ((_R0, k1, np.uint32(ks2 + 1)), (_R1, ks2, np.uint32(k0 + 2)),
             (_R0, k0, np.uint32(k1 + 3)), (_R1, k1, np.uint32(ks2 + 4)),
             (_R0, ks2, np.uint32(k0 + 5)))
    for rots, i0, i1 in sched:
        for r in rots:
            x0 = x0 + x1
            x1 = rotl(x1, r)
            x1 = x1 ^ x0
        x0 = x0 + _i32(i0)
        x1 = x1 + _i32(i1)
    return lax.shift_right_logical(x0 ^ x1, jnp.int32(9))


def _tc_body(avail_ref, x1_ref, x2_ref, out_ref, out1_ref):
    i = pl.program_id(0)
    r0 = _SC_ROWS + i * _RB
    row = lax.broadcasted_iota(jnp.int32, (_RB, _D), 0)
    col = lax.broadcasted_iota(jnp.int32, (_RB, _D), 1)
    flat = (r0 + row) * _D + col

    base = flat * 4
    best = jnp.full((_RB, _D), -1, jnp.int32)
    val = jnp.zeros((_RB, _D), jnp.float32)
    for m in range(4):
        bits = _tf_bits23(_KD1, base + m)
        take = (bits > best) & (avail_ref[m] != 0)
        best = jnp.where(take, bits, best)
        val = jnp.where(take, x1_ref[m], val)
    out1_ref[...] = val

    base = flat * 5
    best = jnp.full((_RB, _D), -1, jnp.int32)
    val2 = jnp.zeros((_RB, _D), jnp.float32)
    for m in range(5):
        bits = _tf_bits23(_KD2, base + m)
        take = (bits > best) & (avail_ref[m] != 0)
        best = jnp.where(take, bits, best)
        val2 = jnp.where(take, x2_ref[m] if m < 4 else val, val2)
    out_ref[...] = val2


def _tc_part(avail_mask, outputs1, outputs2):
    nrows = _B - _SC_ROWS
    off = _SC_ROWS // _RB
    return pl.pallas_call(
        _tc_body,
        grid=(nrows // _RB,),
        in_specs=[
            pl.BlockSpec(memory_space=pltpu.SMEM),
            pl.BlockSpec((4, _RB, _D), lambda i: (0, i + off, 0)),
            pl.BlockSpec((4, _RB, _D), lambda i: (0, i + off, 0)),
        ],
        out_specs=[
            pl.BlockSpec((_RB, _D), lambda i: (i, 0)),
            pl.BlockSpec((_RB, _D), lambda i: (i, 0)),
        ],
        out_shape=[
            jax.ShapeDtypeStruct((nrows, _D), jnp.float32),
            jax.ShapeDtypeStruct((nrows, _D), jnp.float32),
        ],
        compiler_params=pltpu.CompilerParams(
            dimension_semantics=("arbitrary",),
        ),
    )(avail_mask, outputs1, outputs2)


def _sc_chunk_compute(avail_v, in1, in2, out_v, out1_v, r, flat_row):
    av = [avail_v[m] != 0 for m in range(5)]

    def body(t, _):
        lane = jnp.arange(16, dtype=jnp.int32)
        flat = (flat_row * 128 + t) * 16 + lane
        col = t * 16

        base = flat * 4
        best = jnp.full((16,), -1, jnp.int32)
        val = jnp.zeros((16,), jnp.float32)
        for m in range(4):
            bits = _tf_bits23(_KD1, base + m)
            take = (bits > best) & av[m]
            best = jnp.where(take, bits, best)
            val = jnp.where(take, in1[m][r, pl.ds(col, 16)], val)
        out1_v[r, pl.ds(col, 16)] = val

        base = flat * 5
        best = jnp.full((16,), -1, jnp.int32)
        val2 = jnp.zeros((16,), jnp.float32)
        for m in range(5):
            bits = _tf_bits23(_KD2, base + m)
            take = (bits > best) & av[m]
            best = jnp.where(take, bits, best)
            val2 = jnp.where(take, in2[m][r, pl.ds(col, 16)] if m < 4 else val,
                             val2)
        out_v[r, pl.ds(col, 16)] = val2
        return _

    lax.fori_loop(0, _D // 16, body, 0)


def _sc_body(o1_hbm, o2_hbm, avail_hbm, out_hbm, out1_hbm,
             avail_v, in1_v, in2_v, out_v, out1_v):
    wid = lax.axis_index("s") * 2 + lax.axis_index("c")
    rpw = _SC_ROWS // _NW
    row0 = wid * rpw
    pltpu.sync_copy(avail_hbm, avail_v)

    def group(g, _):
        r0 = row0 + g * _RC
        for m in range(4):
            pltpu.sync_copy(o1_hbm.at[m, pl.ds(r0, _RC)], in1_v[m])
            pltpu.sync_copy(o2_hbm.at[m, pl.ds(r0, _RC)], in2_v[m])

        def rowloop(r, c):
            _sc_chunk_compute(avail_v, in1_v, in2_v, out_v, out1_v, r, r0 + r)
            return c

        lax.fori_loop(0, _RC, rowloop, 0)
        pltpu.sync_copy(out1_v, out1_hbm.at[pl.ds(r0, _RC)])
        pltpu.sync_copy(out_v, out_hbm.at[pl.ds(r0, _RC)])
        return _

    lax.fori_loop(0, rpw // _RC, group, 0)


def _sc_part(avail_bcast, outputs1, outputs2):
    mesh = plsc.VectorSubcoreMesh(core_axis_name="c", subcore_axis_name="s")
    run = pl.kernel(
        _sc_body,
        out_type=[
            jax.ShapeDtypeStruct((_SC_ROWS, _D), jnp.float32),
            jax.ShapeDtypeStruct((_SC_ROWS, _D), jnp.float32),
        ],
        mesh=mesh,
        scratch_types=[
            pltpu.VMEM((8, 16), jnp.int32),
            [pltpu.VMEM((_RC, _D), jnp.float32) for _ in range(4)],
            [pltpu.VMEM((_RC, _D), jnp.float32) for _ in range(4)],
            pltpu.VMEM((_RC, _D), jnp.float32),
            pltpu.VMEM((_RC, _D), jnp.float32),
        ],
    )
    return run(outputs1, outputs2, avail_bcast)


def kernel(outputs1, outputs2, available):
    avail_mask = (available != 0.0).astype(jnp.int32)
    avail_bcast = jnp.ones((8, 16), jnp.int32).at[:5].set(avail_mask[:, None])
    sc_out, sc_out1 = _sc_part(avail_bcast, outputs1, outputs2)
    tc_out, tc_out1 = _tc_part(avail_mask, outputs1, outputs2)
    out = jnp.concatenate([sc_out, tc_out], axis=0)
    out1 = jnp.concatenate([sc_out1, tc_out1], axis=0)
    return (out, out1)


# speedup vs baseline: 2.2701x; 1.1802x over previous
import functools

import numpy as np
import jax
import jax.numpy as jnp
from jax import lax
from jax.experimental import pallas as pl
from jax.experimental.pallas import tpu as pltpu
from jax.experimental.pallas import tpu_sc as plsc

_B, _D = 8192, 2048
_RB = 128
_SC_ROWS = 2176
_NW = 32
_RC = 4

_R0 = (13, 15, 26, 6)
_R1 = (17, 29, 16, 24)


def _threefry2x32_np(k0, k1, x0, x1):
    x0 = np.uint32(x0); x1 = np.uint32(x1)
    ks0, ks1 = np.uint32(k0), np.uint32(k1)
    ks2 = np.uint32(ks0 ^ ks1 ^ np.uint32(0x1BD11BDA))
    def rotl(x, r):
        return np.uint32((np.uint32(x << np.uint32(r))) | np.uint32(x >> np.uint32(32 - r)))
    x0 = np.uint32(x0 + ks0); x1 = np.uint32(x1 + ks1)
    sched = [(_R0, ks1, np.uint32(ks2 + 1)), (_R1, ks2, np.uint32(ks0 + 2)),
             (_R0, ks0, np.uint32(ks1 + 3)), (_R1, ks1, np.uint32(ks2 + 4)),
             (_R0, ks2, np.uint32(ks0 + 5))]
    for rots, i0, i1 in sched:
        for r in rots:
            x0 = np.uint32(x0 + x1); x1 = rotl(x1, r); x1 = np.uint32(x1 ^ x0)
        x0 = np.uint32(x0 + i0); x1 = np.uint32(x1 + i1)
    return x0, x1


def _fold_in_42(i):
    with np.errstate(over="ignore"):
        o0, o1 = _threefry2x32_np(0, 42, 0, i)
    return int(np.int32(o0)), int(np.int32(o1))


_KD1 = _fold_in_42(1)
_KD2 = _fold_in_42(2)


def _i32(x):
    return jnp.int32(np.int32(np.uint32(x)))


def _tf_bits23(kd, cnt):
    k0 = np.uint32(np.int32(kd[0]))
    k1 = np.uint32(np.int32(kd[1]))
    ks2 = np.uint32(k0 ^ k1 ^ np.uint32(0x1BD11BDA))

    def rotl(x, r):
        return lax.shift_left(x, jnp.int32(r)) | lax.shift_right_logical(
            x, jnp.int32(32 - r))

    x0 = jnp.full_like(cnt, _i32(k0))
    x1 = cnt + _i32(k1)
    sched = ((_R0, k1, np.uint32(ks2 + 1)), (_R1, ks2, np.uint32(k0 + 2)),
             (_R0, k0, np.uint32(k1 + 3)), (_R1, k1, np.uint32(ks2 + 4)),
             (_R0, ks2, np.uint32(k0 + 5)))
    for rots, i0, i1 in sched:
        for r in rots:
            x0 = x0 + x1
            x1 = rotl(x1, r)
            x1 = x1 ^ x0
        x0 = x0 + _i32(i0)
        x1 = x1 + _i32(i1)
    return lax.shift_right_logical(x0 ^ x1, jnp.int32(9))


def _tc_body(avail_ref, x1_ref, x2_ref, out_ref, out1_ref):
    i = pl.program_id(0)
    r0 = _SC_ROWS + i * _RB
    row = lax.broadcasted_iota(jnp.int32, (_RB, _D), 0)
    col = lax.broadcasted_iota(jnp.int32, (_RB, _D), 1)
    flat = (r0 + row) * _D + col

    base = flat * 4
    best = jnp.full((_RB, _D), -1, jnp.int32)
    val = jnp.zeros((_RB, _D), jnp.float32)
    for m in range(4):
        bits = _tf_bits23(_KD1, base + m)
        take = (bits > best) & (avail_ref[m] != 0)
        best = jnp.where(take, bits, best)
        val = jnp.where(take, x1_ref[m], val)
    out1_ref[...] = val

    base = flat * 5
    best = jnp.full((_RB, _D), -1, jnp.int32)
    val2 = jnp.zeros((_RB, _D), jnp.float32)
    for m in range(5):
        bits = _tf_bits23(_KD2, base + m)
        take = (bits > best) & (avail_ref[m] != 0)
        best = jnp.where(take, bits, best)
        val2 = jnp.where(take, x2_ref[m] if m < 4 else val, val2)
    out_ref[...] = val2


def _tc_part(avail_mask, outputs1, outputs2):
    nrows = _B - _SC_ROWS
    off = _SC_ROWS // _RB
    return pl.pallas_call(
        _tc_body,
        grid=(nrows // _RB,),
        in_specs=[
            pl.BlockSpec(memory_space=pltpu.SMEM),
            pl.BlockSpec((4, _RB, _D), lambda i: (0, i + off, 0)),
            pl.BlockSpec((4, _RB, _D), lambda i: (0, i + off, 0)),
        ],
        out_specs=[
            pl.BlockSpec((_RB, _D), lambda i: (i, 0)),
            pl.BlockSpec((_RB, _D), lambda i: (i, 0)),
        ],
        out_shape=[
            jax.ShapeDtypeStruct((nrows, _D), jnp.float32),
            jax.ShapeDtypeStruct((nrows, _D), jnp.float32),
        ],
        compiler_params=pltpu.CompilerParams(
            dimension_semantics=("arbitrary",),
        ),
    )(avail_mask, outputs1, outputs2)


def _sc_chunk_compute(avail_v, in1, in2, out_v, out1_v, r, flat_row):
    av = [avail_v[m] != 0 for m in range(5)]

    def body(t, _):
        lane = jnp.arange(16, dtype=jnp.int32)
        flat = (flat_row * 128 + t) * 16 + lane
        col = t * 16

        base = flat * 4
        best = jnp.full((16,), -1, jnp.int32)
        val = jnp.zeros((16,), jnp.float32)
        for m in range(4):
            bits = _tf_bits23(_KD1, base + m)
            take = (bits > best) & av[m]
            best = jnp.where(take, bits, best)
            val = jnp.where(take, in1[m][r, pl.ds(col, 16)], val)
        out1_v[r, pl.ds(col, 16)] = val

        base = flat * 5
        best = jnp.full((16,), -1, jnp.int32)
        val2 = jnp.zeros((16,), jnp.float32)
        for m in range(5):
            bits = _tf_bits23(_KD2, base + m)
            take = (bits > best) & av[m]
            best = jnp.where(take, bits, best)
            val2 = jnp.where(take, in2[m][r, pl.ds(col, 16)] if m < 4 else val,
                             val2)
        out_v[r, pl.ds(col, 16)] = val2
        return _

    lax.fori_loop(0, _D // 16, body, 0)


def _sc_body(o1_hbm, o2_hbm, avail_hbm, out_hbm, out1_hbm,
             avail_v, in1_v, in2_v, out_v, out1_v):
    wid = lax.axis_index("s") * 2 + lax.axis_index("c")
    rpw = _SC_ROWS // _NW
    row0 = wid * rpw
    pltpu.sync_copy(avail_hbm, avail_v)

    def group(g, _):
        r0 = row0 + g * _RC
        for m in range(4):
            pltpu.sync_copy(o1_hbm.at[m, pl.ds(r0, _RC)], in1_v[m])
            pltpu.sync_copy(o2_hbm.at[m, pl.ds(r0, _RC)], in2_v[m])

        def rowloop(r, c):
            _sc_chunk_compute(avail_v, in1_v, in2_v, out_v, out1_v, r, r0 + r)
            return c

        lax.fori_loop(0, _RC, rowloop, 0)
        pltpu.sync_copy(out1_v, out1_hbm.at[pl.ds(r0, _RC)])
        pltpu.sync_copy(out_v, out_hbm.at[pl.ds(r0, _RC)])
        return _

    lax.fori_loop(0, rpw // _RC, group, 0)


def _sc_part(avail_bcast, outputs1, outputs2):
    mesh = plsc.VectorSubcoreMesh(core_axis_name="c", subcore_axis_name="s")
    run = pl.kernel(
        _sc_body,
        out_type=[
            jax.ShapeDtypeStruct((_SC_ROWS, _D), jnp.float32),
            jax.ShapeDtypeStruct((_SC_ROWS, _D), jnp.float32),
        ],
        mesh=mesh,
        scratch_types=[
            pltpu.VMEM((8, 16), jnp.int32),
            [pltpu.VMEM((_RC, _D), jnp.float32) for _ in range(4)],
            [pltpu.VMEM((_RC, _D), jnp.float32) for _ in range(4)],
            pltpu.VMEM((_RC, _D), jnp.float32),
            pltpu.VMEM((_RC, _D), jnp.float32),
        ],
    )
    return run(outputs1, outputs2, avail_bcast)


def kernel(outputs1, outputs2, available):
    avail_mask = (available != 0.0).astype(jnp.int32)
    avail_bcast = jnp.ones((8, 16), jnp.int32).at[:5].set(avail_mask[:, None])
    sc_out, sc_out1 = _sc_part(avail_bcast, outputs1, outputs2)
    tc_out, tc_out1 = _tc_part(avail_mask, outputs1, outputs2)
    out = jnp.concatenate([sc_out, tc_out], axis=0)
    out1 = jnp.concatenate([sc_out1, tc_out1], axis=0)
    return (out, out1)


# speedup vs baseline: 2.3822x; 1.0494x over previous
import numpy as np
import jax
import jax.numpy as jnp
from jax import lax
from jax.experimental import pallas as pl
from jax.experimental.pallas import tpu as pltpu
from jax.experimental.pallas import tpu_sc as plsc

_B, _D = 8192, 2048
_RB = 128
_SC_ROWS = 2304
_NW = 32
_RC = 4

_R0 = (13, 15, 26, 6)
_R1 = (17, 29, 16, 24)


def _threefry2x32_np(k0, k1, x0, x1):
    x0 = np.uint32(x0); x1 = np.uint32(x1)
    ks0, ks1 = np.uint32(k0), np.uint32(k1)
    ks2 = np.uint32(ks0 ^ ks1 ^ np.uint32(0x1BD11BDA))
    def rotl(x, r):
        return np.uint32((np.uint32(x << np.uint32(r))) | np.uint32(x >> np.uint32(32 - r)))
    x0 = np.uint32(x0 + ks0); x1 = np.uint32(x1 + ks1)
    sched = [(_R0, ks1, np.uint32(ks2 + 1)), (_R1, ks2, np.uint32(ks0 + 2)),
             (_R0, ks0, np.uint32(ks1 + 3)), (_R1, ks1, np.uint32(ks2 + 4)),
             (_R0, ks2, np.uint32(ks0 + 5))]
    for rots, i0, i1 in sched:
        for r in rots:
            x0 = np.uint32(x0 + x1); x1 = rotl(x1, r); x1 = np.uint32(x1 ^ x0)
        x0 = np.uint32(x0 + i0); x1 = np.uint32(x1 + i1)
    return x0, x1


def _fold_in_42(i):
    with np.errstate(over="ignore"):
        o0, o1 = _threefry2x32_np(0, 42, 0, i)
    return int(np.int32(o0)), int(np.int32(o1))


_KD1 = _fold_in_42(1)
_KD2 = _fold_in_42(2)


def _i32(x):
    return jnp.int32(np.int32(np.uint32(x)))


def _tf_bits23(kd, cnt):
    k0 = np.uint32(np.int32(kd[0]))
    k1 = np.uint32(np.int32(kd[1]))
    ks2 = np.uint32(k0 ^ k1 ^ np.uint32(0x1BD11BDA))

    def rotl(x, r):
        return lax.shift_left(x, jnp.int32(r)) | lax.shift_right_logical(
            x, jnp.int32(32 - r))

    x0 = jnp.full_like(cnt, _i32(k0))
    x1 = cnt + _i32(k1)
    sched = ((_R0, k1, np.uint32(ks2 + 1)), (_R1, ks2, np.uint32(k0 + 2)),
             (_R0, k0, np.uint32(k1 + 3)), (_R1, k1, np.uint32(ks2 + 4)),
             (_R0, ks2, np.uint32(k0 + 5)))
    for rots, i0, i1 in sched:
        for r in rots:
            x0 = x0 + x1
            x1 = rotl(x1, r)
            x1 = x1 ^ x0
        x0 = x0 + _i32(i0)
        x1 = x1 + _i32(i1)
    return lax.shift_right_logical(x0 ^ x1, jnp.int32(9))


def _tc_body(avail_ref, x1_ref, x2_ref, out_ref, out1_ref):
    i = pl.program_id(0)
    r0 = _SC_ROWS + i * _RB
    row = lax.broadcasted_iota(jnp.int32, (_RB, _D), 0)
    col = lax.broadcasted_iota(jnp.int32, (_RB, _D), 1)
    flat = (r0 + row) * _D + col

    base = flat * 4
    best = jnp.full((_RB, _D), -1, jnp.int32)
    val = jnp.zeros((_RB, _D), jnp.float32)
    for m in range(4):
        bits = _tf_bits23(_KD1, base + m)
        take = (bits > best) & (avail_ref[m] != 0)
        best = jnp.where(take, bits, best)
        val = jnp.where(take, x1_ref[m], val)
    out1_ref[...] = val

    base = flat * 5
    best = jnp.full((_RB, _D), -1, jnp.int32)
    val2 = jnp.zeros((_RB, _D), jnp.float32)
    for m in range(5):
        bits = _tf_bits23(_KD2, base + m)
        take = (bits > best) & (avail_ref[m] != 0)
        best = jnp.where(take, bits, best)
        val2 = jnp.where(take, x2_ref[m] if m < 4 else val, val2)
    out_ref[...] = val2


def _tc_part(avail_mask, outputs1, outputs2):
    nrows = _B - _SC_ROWS
    off = _SC_ROWS // _RB
    return pl.pallas_call(
        _tc_body,
        grid=(nrows // _RB,),
        in_specs=[
            pl.BlockSpec(memory_space=pltpu.SMEM),
            pl.BlockSpec((4, _RB, _D), lambda i: (0, i + off, 0)),
            pl.BlockSpec((4, _RB, _D), lambda i: (0, i + off, 0)),
        ],
        out_specs=[
            pl.BlockSpec((_RB, _D), lambda i: (i + off, 0)),
            pl.BlockSpec((_RB, _D), lambda i: (i + off, 0)),
        ],
        out_shape=[
            jax.ShapeDtypeStruct((_B, _D), jnp.float32),
            jax.ShapeDtypeStruct((_B, _D), jnp.float32),
        ],
        compiler_params=pltpu.CompilerParams(
            dimension_semantics=("arbitrary",),
        ),
    )(avail_mask, outputs1, outputs2)


def _sc_chunk_compute(avail_v, in1, in2, out_v, out1_v, r, flat_row):
    av = [avail_v[m] != 0 for m in range(5)]

    def body(t, _):
        lane = jnp.arange(16, dtype=jnp.int32)
        flat = (flat_row * 128 + t) * 16 + lane
        col = t * 16

        base = flat * 4
        best = jnp.full((16,), -1, jnp.int32)
        val = jnp.zeros((16,), jnp.float32)
        for m in range(4):
            bits = _tf_bits23(_KD1, base + m)
            take = (bits > best) & av[m]
            best = jnp.where(take, bits, best)
            val = jnp.where(take, in1[m][r, pl.ds(col, 16)], val)
        out1_v[r, pl.ds(col, 16)] = val

        base = flat * 5
        best = jnp.full((16,), -1, jnp.int32)
        val2 = jnp.zeros((16,), jnp.float32)
        for m in range(5):
            bits = _tf_bits23(_KD2, base + m)
            take = (bits > best) & av[m]
            best = jnp.where(take, bits, best)
            val2 = jnp.where(take, in2[m][r, pl.ds(col, 16)] if m < 4 else val,
                             val2)
        out_v[r, pl.ds(col, 16)] = val2
        return _

    lax.fori_loop(0, _D // 16, body, 0)


def _sc_body(o1_hbm, o2_hbm, avail_hbm, out_hbm, out1_hbm,
             avail_v, in1_v, in2_v, out_v, out1_v):
    wid = lax.axis_index("s") * 2 + lax.axis_index("c")
    rpw = _SC_ROWS // _NW
    row0 = wid * rpw
    pltpu.sync_copy(avail_hbm, avail_v)

    def group(g, _):
        r0 = row0 + g * _RC
        for m in range(4):
            pltpu.sync_copy(o1_hbm.at[m, pl.ds(r0, _RC)], in1_v[m])
            pltpu.sync_copy(o2_hbm.at[m, pl.ds(r0, _RC)], in2_v[m])

        def rowloop(r, c):
            _sc_chunk_compute(avail_v, in1_v, in2_v, out_v, out1_v, r, r0 + r)
            return c

        lax.fori_loop(0, _RC, rowloop, 0)
        pltpu.sync_copy(out1_v, out1_hbm.at[pl.ds(r0, _RC)])
        pltpu.sync_copy(out_v, out_hbm.at[pl.ds(r0, _RC)])
        return _

    lax.fori_loop(0, rpw // _RC, group, 0)


def _sc_part(avail_bcast, outputs1, outputs2):
    mesh = plsc.VectorSubcoreMesh(core_axis_name="c", subcore_axis_name="s")
    run = pl.kernel(
        _sc_body,
        out_type=[
            jax.ShapeDtypeStruct((_SC_ROWS, _D), jnp.float32),
            jax.ShapeDtypeStruct((_SC_ROWS, _D), jnp.float32),
        ],
        mesh=mesh,
        scratch_types=[
            pltpu.VMEM((8, 16), jnp.int32),
            [pltpu.VMEM((_RC, _D), jnp.float32) for _ in range(4)],
            [pltpu.VMEM((_RC, _D), jnp.float32) for _ in range(4)],
            pltpu.VMEM((_RC, _D), jnp.float32),
            pltpu.VMEM((_RC, _D), jnp.float32),
        ],
    )
    return run(outputs1, outputs2, avail_bcast)


def kernel(outputs1, outputs2, available):
    avail_mask = (available != 0.0).astype(jnp.int32)
    avail_bcast = jnp.ones((8, 16), jnp.int32).at[:5].set(avail_mask[:, None])
    sc_out, sc_out1 = _sc_part(avail_bcast, outputs1, outputs2)
    tc_out, tc_out1 = _tc_part(avail_mask, outputs1, outputs2)
    out = lax.dynamic_update_slice(tc_out, sc_out, (0, 0))
    out1 = lax.dynamic_update_slice(tc_out1, sc_out1, (0, 0))
    return (out, out1)


# speedup vs baseline: 2.4091x; 1.0113x over previous
import numpy as np
import jax
import jax.numpy as jnp
from jax import lax
from jax.experimental import pallas as pl
from jax.experimental.pallas import tpu as pltpu
from jax.experimental.pallas import tpu_sc as plsc

_B, _D = 8192, 2048
_RB = 256
_SC_ROWS = 2304
_NW = 32
_RC = 4

_R0 = (13, 15, 26, 6)
_R1 = (17, 29, 16, 24)


def _threefry2x32_np(k0, k1, x0, x1):
    x0 = np.uint32(x0); x1 = np.uint32(x1)
    ks0, ks1 = np.uint32(k0), np.uint32(k1)
    ks2 = np.uint32(ks0 ^ ks1 ^ np.uint32(0x1BD11BDA))
    def rotl(x, r):
        return np.uint32((np.uint32(x << np.uint32(r))) | np.uint32(x >> np.uint32(32 - r)))
    x0 = np.uint32(x0 + ks0); x1 = np.uint32(x1 + ks1)
    sched = [(_R0, ks1, np.uint32(ks2 + 1)), (_R1, ks2, np.uint32(ks0 + 2)),
             (_R0, ks0, np.uint32(ks1 + 3)), (_R1, ks1, np.uint32(ks2 + 4)),
             (_R0, ks2, np.uint32(ks0 + 5))]
    for rots, i0, i1 in sched:
        for r in rots:
            x0 = np.uint32(x0 + x1); x1 = rotl(x1, r); x1 = np.uint32(x1 ^ x0)
        x0 = np.uint32(x0 + i0); x1 = np.uint32(x1 + i1)
    return x0, x1


def _fold_in_42(i):
    with np.errstate(over="ignore"):
        o0, o1 = _threefry2x32_np(0, 42, 0, i)
    return int(np.int32(o0)), int(np.int32(o1))


_KD1 = _fold_in_42(1)
_KD2 = _fold_in_42(2)


def _i32(x):
    return jnp.int32(np.int32(np.uint32(x)))


def _tf_bits23(kd, cnt):
    k0 = np.uint32(np.int32(kd[0]))
    k1 = np.uint32(np.int32(kd[1]))
    ks2 = np.uint32(k0 ^ k1 ^ np.uint32(0x1BD11BDA))

    def rotl(x, r):
        return lax.shift_left(x, jnp.int32(r)) | lax.shift_right_logical(
            x, jnp.int32(32 - r))

    x0 = jnp.full_like(cnt, _i32(k0))
    x1 = cnt + _i32(k1)
    sched = ((_R0, k1, np.uint32(ks2 + 1)), (_R1, ks2, np.uint32(k0 + 2)),
             (_R0, k0, np.uint32(k1 + 3)), (_R1, k1, np.uint32(ks2 + 4)),
             (_R0, ks2, np.uint32(k0 + 5)))
    for rots, i0, i1 in sched:
        for r in rots:
            x0 = x0 + x1
            x1 = rotl(x1, r)
            x1 = x1 ^ x0
        x0 = x0 + _i32(i0)
        x1 = x1 + _i32(i1)
    return lax.shift_right_logical(x0 ^ x1, jnp.int32(9))


def _tc_body(avail_ref, x1_ref, x2_ref, out_ref, out1_ref):
    i = pl.program_id(0)
    r0 = _SC_ROWS + i * _RB
    row = lax.broadcasted_iota(jnp.int32, (_RB, _D), 0)
    col = lax.broadcasted_iota(jnp.int32, (_RB, _D), 1)
    flat = (r0 + row) * _D + col

    base = flat * 4
    best = jnp.full((_RB, _D), -1, jnp.int32)
    val = jnp.zeros((_RB, _D), jnp.float32)
    for m in range(4):
        bits = _tf_bits23(_KD1, base + m)
        take = (bits > best) & (avail_ref[m] != 0)
        best = jnp.where(take, bits, best)
        val = jnp.where(take, x1_ref[m], val)
    out1_ref[...] = val

    base = flat * 5
    best = jnp.full((_RB, _D), -1, jnp.int32)
    val2 = jnp.zeros((_RB, _D), jnp.float32)
    for m in range(5):
        bits = _tf_bits23(_KD2, base + m)
        take = (bits > best) & (avail_ref[m] != 0)
        best = jnp.where(take, bits, best)
        val2 = jnp.where(take, x2_ref[m] if m < 4 else val, val2)
    out_ref[...] = val2


def _tc_part(avail_mask, outputs1, outputs2):
    nrows = _B - _SC_ROWS
    off = _SC_ROWS // _RB
    return pl.pallas_call(
        _tc_body,
        grid=(nrows // _RB,),
        in_specs=[
            pl.BlockSpec(memory_space=pltpu.SMEM),
            pl.BlockSpec((4, _RB, _D), lambda i: (0, i + off, 0)),
            pl.BlockSpec((4, _RB, _D), lambda i: (0, i + off, 0)),
        ],
        out_specs=[
            pl.BlockSpec((_RB, _D), lambda i: (i + off, 0)),
            pl.BlockSpec((_RB, _D), lambda i: (i + off, 0)),
        ],
        out_shape=[
            jax.ShapeDtypeStruct((_B, _D), jnp.float32),
            jax.ShapeDtypeStruct((_B, _D), jnp.float32),
        ],
        compiler_params=pltpu.CompilerParams(
            dimension_semantics=("arbitrary",),
        ),
    )(avail_mask, outputs1, outputs2)


def _sc_chunk_compute(avail_v, in1, in2, out_v, out1_v, r, flat_row):
    av = [avail_v[m] != 0 for m in range(5)]

    def body(t, _):
        lane = jnp.arange(16, dtype=jnp.int32)
        flat = (flat_row * 128 + t) * 16 + lane
        col = t * 16

        base = flat * 4
        best = jnp.full((16,), -1, jnp.int32)
        val = jnp.zeros((16,), jnp.float32)
        for m in range(4):
            bits = _tf_bits23(_KD1, base + m)
            take = (bits > best) & av[m]
            best = jnp.where(take, bits, best)
            val = jnp.where(take, in1[m][r, pl.ds(col, 16)], val)
        out1_v[r, pl.ds(col, 16)] = val

        base = flat * 5
        best = jnp.full((16,), -1, jnp.int32)
        val2 = jnp.zeros((16,), jnp.float32)
        for m in range(5):
            bits = _tf_bits23(_KD2, base + m)
            take = (bits > best) & av[m]
            best = jnp.where(take, bits, best)
            val2 = jnp.where(take, in2[m][r, pl.ds(col, 16)] if m < 4 else val,
                             val2)
        out_v[r, pl.ds(col, 16)] = val2
        return _

    lax.fori_loop(0, _D // 16, body, 0)


def _sc_body(o1_hbm, o2_hbm, avail_hbm, out_hbm, out1_hbm,
             avail_v, in1_v, in2_v, out_v, out1_v):
    wid = lax.axis_index("s") * 2 + lax.axis_index("c")
    rpw = _SC_ROWS // _NW
    row0 = wid * rpw
    pltpu.sync_copy(avail_hbm, avail_v)

    def group(g, _):
        r0 = row0 + g * _RC
        for m in range(4):
            pltpu.sync_copy(o1_hbm.at[m, pl.ds(r0, _RC)], in1_v[m])
            pltpu.sync_copy(o2_hbm.at[m, pl.ds(r0, _RC)], in2_v[m])

        def rowloop(r, c):
            _sc_chunk_compute(avail_v, in1_v, in2_v, out_v, out1_v, r, r0 + r)
            return c

        lax.fori_loop(0, _RC, rowloop, 0)
        pltpu.sync_copy(out1_v, out1_hbm.at[pl.ds(r0, _RC)])
        pltpu.sync_copy(out_v, out_hbm.at[pl.ds(r0, _RC)])
        return _

    lax.fori_loop(0, rpw // _RC, group, 0)


def _sc_part(avail_bcast, outputs1, outputs2):
    mesh = plsc.VectorSubcoreMesh(core_axis_name="c", subcore_axis_name="s")
    run = pl.kernel(
        _sc_body,
        out_type=[
            jax.ShapeDtypeStruct((_SC_ROWS, _D), jnp.float32),
            jax.ShapeDtypeStruct((_SC_ROWS, _D), jnp.float32),
        ],
        mesh=mesh,
        scratch_types=[
            pltpu.VMEM((8, 16), jnp.int32),
            [pltpu.VMEM((_RC, _D), jnp.float32) for _ in range(4)],
            [pltpu.VMEM((_RC, _D), jnp.float32) for _ in range(4)],
            pltpu.VMEM((_RC, _D), jnp.float32),
            pltpu.VMEM((_RC, _D), jnp.float32),
        ],
    )
    return run(outputs1, outputs2, avail_bcast)


def kernel(outputs1, outputs2, available):
    avail_mask = (available != 0.0).astype(jnp.int32)
    avail_bcast = jnp.ones((8, 16), jnp.int32).at[:5].set(avail_mask[:, None])
    sc_out, sc_out1 = _sc_part(avail_bcast, outputs1, outputs2)
    tc_out, tc_out1 = _tc_part(avail_mask, outputs1, outputs2)
    out = lax.dynamic_update_slice(tc_out, sc_out, (0, 0))
    out1 = lax.dynamic_update_slice(tc_out1, sc_out1, (0, 0))
    return (out, out1)


# speedup vs baseline: 2.6045x; 1.0811x over previous
import numpy as np
import jax
import jax.numpy as jnp
from jax import lax
from jax.experimental import pallas as pl
from jax.experimental.pallas import tpu as pltpu
from jax.experimental.pallas import tpu_sc as plsc

_B, _D = 8192, 2048
_RB = 256
_SC_ROWS = 2304
_NW = 32
_RC = 2

_R0 = (13, 15, 26, 6)
_R1 = (17, 29, 16, 24)


def _threefry2x32_np(k0, k1, x0, x1):
    x0 = np.uint32(x0); x1 = np.uint32(x1)
    ks0, ks1 = np.uint32(k0), np.uint32(k1)
    ks2 = np.uint32(ks0 ^ ks1 ^ np.uint32(0x1BD11BDA))
    def rotl(x, r):
        return np.uint32((np.uint32(x << np.uint32(r))) | np.uint32(x >> np.uint32(32 - r)))
    x0 = np.uint32(x0 + ks0); x1 = np.uint32(x1 + ks1)
    sched = [(_R0, ks1, np.uint32(ks2 + 1)), (_R1, ks2, np.uint32(ks0 + 2)),
             (_R0, ks0, np.uint32(ks1 + 3)), (_R1, ks1, np.uint32(ks2 + 4)),
             (_R0, ks2, np.uint32(ks0 + 5))]
    for rots, i0, i1 in sched:
        for r in rots:
            x0 = np.uint32(x0 + x1); x1 = rotl(x1, r); x1 = np.uint32(x1 ^ x0)
        x0 = np.uint32(x0 + i0); x1 = np.uint32(x1 + i1)
    return x0, x1


def _fold_in_42(i):
    with np.errstate(over="ignore"):
        o0, o1 = _threefry2x32_np(0, 42, 0, i)
    return int(np.int32(o0)), int(np.int32(o1))


_KD1 = _fold_in_42(1)
_KD2 = _fold_in_42(2)


def _i32(x):
    return jnp.int32(np.int32(np.uint32(x)))


def _tf_bits23(kd, cnt):
    k0 = np.uint32(np.int32(kd[0]))
    k1 = np.uint32(np.int32(kd[1]))
    ks2 = np.uint32(k0 ^ k1 ^ np.uint32(0x1BD11BDA))

    def rotl(x, r):
        return lax.shift_left(x, jnp.int32(r)) | lax.shift_right_logical(
            x, jnp.int32(32 - r))

    x0 = jnp.full_like(cnt, _i32(k0))
    x1 = cnt + _i32(k1)
    sched = ((_R0, k1, np.uint32(ks2 + 1)), (_R1, ks2, np.uint32(k0 + 2)),
             (_R0, k0, np.uint32(k1 + 3)), (_R1, k1, np.uint32(ks2 + 4)),
             (_R0, ks2, np.uint32(k0 + 5)))
    for rots, i0, i1 in sched:
        for r in rots:
            x0 = x0 + x1
            x1 = rotl(x1, r)
            x1 = x1 ^ x0
        x0 = x0 + _i32(i0)
        x1 = x1 + _i32(i1)
    return lax.shift_right_logical(x0 ^ x1, jnp.int32(9))


def _tc_body(avail_ref, x1_ref, x2_ref, out_ref, out1_ref):
    i = pl.program_id(0)
    r0 = _SC_ROWS + i * _RB
    row = lax.broadcasted_iota(jnp.int32, (_RB, _D), 0)
    col = lax.broadcasted_iota(jnp.int32, (_RB, _D), 1)
    flat = (r0 + row) * _D + col

    base = flat * 4
    best = jnp.full((_RB, _D), -1, jnp.int32)
    val = jnp.zeros((_RB, _D), jnp.float32)
    for m in range(4):
        bits = _tf_bits23(_KD1, base + m)
        take = (bits > best) & (avail_ref[m] != 0)
        best = jnp.where(take, bits, best)
        val = jnp.where(take, x1_ref[m], val)
    out1_ref[...] = val

    base = flat * 5
    best = jnp.full((_RB, _D), -1, jnp.int32)
    val2 = jnp.zeros((_RB, _D), jnp.float32)
    for m in range(5):
        bits = _tf_bits23(_KD2, base + m)
        take = (bits > best) & (avail_ref[m] != 0)
        best = jnp.where(take, bits, best)
        val2 = jnp.where(take, x2_ref[m] if m < 4 else val, val2)
    out_ref[...] = val2


def _tc_part(avail_mask, outputs1, outputs2):
    nrows = _B - _SC_ROWS
    off = _SC_ROWS // _RB
    return pl.pallas_call(
        _tc_body,
        grid=(nrows // _RB,),
        in_specs=[
            pl.BlockSpec(memory_space=pltpu.SMEM),
            pl.BlockSpec((4, _RB, _D), lambda i: (0, i + off, 0)),
            pl.BlockSpec((4, _RB, _D), lambda i: (0, i + off, 0)),
        ],
        out_specs=[
            pl.BlockSpec((_RB, _D), lambda i: (i + off, 0)),
            pl.BlockSpec((_RB, _D), lambda i: (i + off, 0)),
        ],
        out_shape=[
            jax.ShapeDtypeStruct((_B, _D), jnp.float32),
            jax.ShapeDtypeStruct((_B, _D), jnp.float32),
        ],
        compiler_params=pltpu.CompilerParams(
            dimension_semantics=("arbitrary",),
        ),
    )(avail_mask, outputs1, outputs2)


def _sc_chunk_compute(avail_v, in1, in2, out_v, out1_v, r, flat_row):
    av = [avail_v[m] != 0 for m in range(5)]

    def body(t, _):
        lane = jnp.arange(16, dtype=jnp.int32)
        flat = (flat_row * 128 + t) * 16 + lane
        col = t * 16

        base = flat * 4
        best = jnp.full((16,), -1, jnp.int32)
        val = jnp.zeros((16,), jnp.float32)
        for m in range(4):
            bits = _tf_bits23(_KD1, base + m)
            take = (bits > best) & av[m]
            best = jnp.where(take, bits, best)
            val = jnp.where(take, in1[m][r, pl.ds(col, 16)], val)
        out1_v[r, pl.ds(col, 16)] = val

        base = flat * 5
        best = jnp.full((16,), -1, jnp.int32)
        val2 = jnp.zeros((16,), jnp.float32)
        for m in range(5):
            bits = _tf_bits23(_KD2, base + m)
            take = (bits > best) & av[m]
            best = jnp.where(take, bits, best)
            val2 = jnp.where(take, in2[m][r, pl.ds(col, 16)] if m < 4 else val,
                             val2)
        out_v[r, pl.ds(col, 16)] = val2
        return _

    lax.fori_loop(0, _D // 16, body, 0)


def _sc_body(o1_hbm, o2_hbm, avail_hbm, out_hbm, out1_hbm,
             avail_v, in1_a, in2_a, in1_b, in2_b,
             out_a, out1_a, out_b, out1_b,
             sem_a, sem_b, sem_oa, sem_ob):
    wid = lax.axis_index("s") * 2 + lax.axis_index("c")
    rpw = _SC_ROWS // _NW
    row0 = wid * rpw
    ngroups = rpw // _RC
    pltpu.sync_copy(avail_hbm, avail_v)

    bufsets = ((in1_a, in2_a, out_a, out1_a, sem_a, sem_oa),
               (in1_b, in2_b, out_b, out1_b, sem_b, sem_ob))

    def fire_in(g, in1, in2, sem):
        r0 = row0 + g * _RC
        for m in range(4):
            pltpu.async_copy(o1_hbm.at[m, pl.ds(r0, _RC)], in1[m], sem)
            pltpu.async_copy(o2_hbm.at[m, pl.ds(r0, _RC)], in2[m], sem)

    def drain_in(g, in1, in2, sem):
        r0 = row0 + g * _RC
        for m in range(4):
            pltpu.make_async_copy(o1_hbm.at[m, pl.ds(r0, _RC)], in1[m],
                                  sem).wait()
            pltpu.make_async_copy(o2_hbm.at[m, pl.ds(r0, _RC)], in2[m],
                                  sem).wait()

    fire_in(0, in1_a, in2_a, sem_a)
    fire_in(1, in1_b, in2_b, sem_b)

    def pair(gg, _):
        for b in range(2):
            in1, in2, out_v, out1_v, sem, sem_o = bufsets[b]
            g = 2 * gg + b
            drain_in(g, in1, in2, sem)

            @pl.when(g >= 2)
            def _drain_out():
                r0p = row0 + (g - 2) * _RC
                pltpu.make_async_copy(out1_v, out1_hbm.at[pl.ds(r0p, _RC)],
                                      sem_o).wait()
                pltpu.make_async_copy(out_v, out_hbm.at[pl.ds(r0p, _RC)],
                                      sem_o).wait()

            r0 = row0 + g * _RC

            def rowloop(r, c):
                _sc_chunk_compute(avail_v, in1, in2, out_v, out1_v, r, r0 + r)
                return c

            lax.fori_loop(0, _RC, rowloop, 0)
            pltpu.async_copy(out1_v, out1_hbm.at[pl.ds(r0, _RC)], sem_o)
            pltpu.async_copy(out_v, out_hbm.at[pl.ds(r0, _RC)], sem_o)

            @pl.when(g + 2 < ngroups)
            def _fire():
                fire_in(g + 2, in1, in2, sem)
        return _

    lax.fori_loop(0, ngroups // 2, pair, 0)

    for b in range(2):
        in1, in2, out_v, out1_v, sem, sem_o = bufsets[b]
        g = ngroups - 2 + b
        r0 = row0 + g * _RC
        pltpu.make_async_copy(out1_v, out1_hbm.at[pl.ds(r0, _RC)],
                              sem_o).wait()
        pltpu.make_async_copy(out_v, out_hbm.at[pl.ds(r0, _RC)],
                              sem_o).wait()


def _sc_part(avail_bcast, outputs1, outputs2):
    mesh = plsc.VectorSubcoreMesh(core_axis_name="c", subcore_axis_name="s")
    run = pl.kernel(
        _sc_body,
        out_type=[
            jax.ShapeDtypeStruct((_SC_ROWS, _D), jnp.float32),
            jax.ShapeDtypeStruct((_SC_ROWS, _D), jnp.float32),
        ],
        mesh=mesh,
        scratch_types=[
            pltpu.VMEM((8, 16), jnp.int32),
            [pltpu.VMEM((_RC, _D), jnp.float32) for _ in range(4)],
            [pltpu.VMEM((_RC, _D), jnp.float32) for _ in range(4)],
            [pltpu.VMEM((_RC, _D), jnp.float32) for _ in range(4)],
            [pltpu.VMEM((_RC, _D), jnp.float32) for _ in range(4)],
            pltpu.VMEM((_RC, _D), jnp.float32),
            pltpu.VMEM((_RC, _D), jnp.float32),
            pltpu.VMEM((_RC, _D), jnp.float32),
            pltpu.VMEM((_RC, _D), jnp.float32),
            pltpu.SemaphoreType.DMA,
            pltpu.SemaphoreType.DMA,
            pltpu.SemaphoreType.DMA,
            pltpu.SemaphoreType.DMA,
        ],
    )
    return run(outputs1, outputs2, avail_bcast)


def kernel(outputs1, outputs2, available):
    avail_mask = (available != 0.0).astype(jnp.int32)
    avail_bcast = jnp.ones((8, 16), jnp.int32).at[:5].set(avail_mask[:, None])
    sc_out, sc_out1 = _sc_part(avail_bcast, outputs1, outputs2)
    tc_out, tc_out1 = _tc_part(avail_mask, outputs1, outputs2)
    out = lax.dynamic_update_slice(tc_out, sc_out, (0, 0))
    out1 = lax.dynamic_update_slice(tc_out1, sc_out1, (0, 0))
    return (out, out1)
